# baseline (speedup 1.0000x reference)
import jax
import jax.numpy as jnp
from jax.experimental import pallas as pl
from jax.experimental.pallas import tpu as pltpu
from jax.experimental.pallas import tpu_sc as plsc

_BLKR = 8
_BLKC = 1024
_VEC = 16


def kernel(inputs, embeddings):
    batch, seq_len, dim = inputs.shape
    rows = batch * seq_len
    nseq_r = seq_len // _BLKR
    flat = inputs.reshape(rows, dim)
    mesh = plsc.VectorSubcoreMesh(core_axis_name="c", subcore_axis_name="s")

    @pl.kernel(
        out_type=jax.ShapeDtypeStruct((rows, dim), inputs.dtype),
        mesh=mesh,
    )
    def sc_add(x_hbm, e_hbm, o_hbm):
        def body(x_vmem, e_vmem, o_vmem):
            @pl.loop(0, _BLKR)
            def _(r):
                @pl.loop(0, _BLKC, step=_VEC)
                def _(c):
                    slc = (pl.ds(r, 1), pl.ds(c, _VEC))
                    o_vmem.at[*slc][...] = x_vmem.at[*slc][...] + e_vmem.at[*slc][...]

        pltpu.emit_pipeline(
            body,
            grid=(rows // _BLKR, dim // _BLKC),
            in_specs=[
                pl.BlockSpec((_BLKR, _BLKC), index_map=lambda i, j: (i, j)),
                pl.BlockSpec((_BLKR, _BLKC), index_map=lambda i, j: (i % nseq_r, j)),
            ],
            out_specs=[pl.BlockSpec((_BLKR, _BLKC), index_map=lambda i, j: (i, j))],
            core_axis_name=("c", "s"),
            dimension_semantics=(pltpu.PARALLEL, pltpu.PARALLEL),
        )(x_hbm, e_hbm, o_hbm)

    return sc_add(flat, embeddings).reshape(batch, seq_len, dim)


# speedup vs baseline: 1.8619x; 1.8619x over previous
import jax
import jax.numpy as jnp
from jax.experimental import pallas as pl
from jax.experimental.pallas import tpu as pltpu
from jax.experimental.pallas import tpu_sc as plsc

_BLKR = 8
_BLKC = 1024
_VEC = 16


def kernel(inputs, embeddings):
    batch, seq_len, dim = inputs.shape
    rows = batch * seq_len
    nseq_r = seq_len // _BLKR
    flat = inputs.reshape(rows, dim)
    mesh = plsc.VectorSubcoreMesh(core_axis_name="c", subcore_axis_name="s")

    @pl.kernel(
        out_type=jax.ShapeDtypeStruct((rows, dim), inputs.dtype),
        mesh=mesh,
    )
    def sc_add(x_hbm, e_hbm, o_hbm):
        def body(x_vmem, e_vmem, o_vmem):
            @pl.loop(0, _BLKR)
            def _(r):
                @plsc.parallel_loop(0, _BLKC, step=_VEC, unroll=8)
                def _(c):
                    slc = (pl.ds(r, 1), pl.ds(c, _VEC))
                    o_vmem.at[*slc][...] = x_vmem.at[*slc][...] + e_vmem.at[*slc][...]

        pltpu.emit_pipeline(
            body,
            grid=(rows // _BLKR, dim // _BLKC),
            in_specs=[
                pl.BlockSpec((_BLKR, _BLKC), index_map=lambda i, j: (i, j)),
                pl.BlockSpec((_BLKR, _BLKC), index_map=lambda i, j: (i % nseq_r, j)),
            ],
            out_specs=[pl.BlockSpec((_BLKR, _BLKC), index_map=lambda i, j: (i, j))],
            core_axis_name=("c", "s"),
            dimension_semantics=(pltpu.PARALLEL, pltpu.PARALLEL),
        )(x_hbm, e_hbm, o_hbm)

    return sc_add(flat, embeddings).reshape(batch, seq_len, dim)


# speedup vs baseline: 3.9464x; 2.1196x over previous
import jax
import jax.numpy as jnp
from jax.experimental import pallas as pl
from jax.experimental.pallas import tpu as pltpu


def _add_body(x_ref, e_ref, o_ref):
    o_ref[...] = x_ref[...] + e_ref[...][None, :, :]


def kernel(inputs, embeddings):
    batch, seq_len, dim = inputs.shape
    blk = 512
    grid = (seq_len // blk,)
    emb = pltpu.with_memory_space_constraint(
        embeddings[:seq_len], pltpu.MemorySpace.HBM
    )
    return pl.pallas_call(
        _add_body,
        grid=grid,
        in_specs=[
            pl.BlockSpec((batch, blk, dim), lambda i: (0, i, 0)),
            pl.BlockSpec((blk, dim), lambda i: (i, 0)),
        ],
        out_specs=pl.BlockSpec((batch, blk, dim), lambda i: (0, i, 0)),
        out_shape=jax.ShapeDtypeStruct((batch, seq_len, dim), inputs.dtype),
    )(inputs, emb)
